# baseline (device time: 118840 ns/iter reference)
import jax
import jax.numpy as jnp
from jax import lax
from jax.experimental import pallas as pl
from jax.experimental.pallas import tpu as pltpu

N_DEV = 16
P = 4
N_MACRO = 4
SUB = 128
BLK = 256


def kernel(x, w_mat, scale_x, scale_w):
    m_total, k = x.shape
    _, n = w_mat.shape
    m_per = m_total // N_DEV
    half = n // 2
    g_rows = P * m_per

    def body(x_ref, w_ref, sx_ref, sw_ref, out_ref,
             xg_ref, comm_a, stage, in_up, in_dn,
             a_send, a_recv, up_send, up_recv, dn_send, dn_recv):
        my = lax.axis_index("i")
        q = lax.rem(my, P)
        r = my // P
        base = my - q
        pnext = base + lax.rem(q + 1, P)
        pprev = base + lax.rem(q + P - 1, P)

        for g in range(P):
            for rb in range(P):
                c = P * rb + g
                xg_ref[pl.ds((g * P + rb) * m_per, m_per), :] = (
                    x_ref[pl.ds(c * m_per, m_per), :]
                )

        def pgroup(g, c0, width):
            return jnp.dot(
                xg_ref[pl.ds(g * g_rows, g_rows), :],
                w_ref[:, c0:c0 + width],
                preferred_element_type=jnp.float32,
            )

        def mk_a(idx, s, dest):
            return pltpu.make_async_remote_copy(
                src_ref=comm_a.at[idx, s],
                dst_ref=comm_a.at[idx, s + 1],
                send_sem=a_send.at[idx, s],
                recv_sem=a_recv.at[idx, s + 1],
                device_id=(dest,),
                device_id_type=pl.DeviceIdType.MESH,
            )

        def mk_up(t8, o):
            return pltpu.make_async_remote_copy(
                src_ref=in_up.at[t8, o], dst_ref=in_up.at[t8, o],
                send_sem=up_send.at[t8, o], recv_sem=up_recv.at[t8, o],
                device_id=(my + P,), device_id_type=pl.DeviceIdType.MESH,
            )

        def mk_dn(t8, o):
            return pltpu.make_async_remote_copy(
                src_ref=in_dn.at[t8, o], dst_ref=in_dn.at[t8, o],
                send_sem=dn_send.at[t8, o], recv_sem=dn_recv.at[t8, o],
                device_id=(my - P,), device_id_type=pl.DeviceIdType.MESH,
            )

        scale = sx_ref[0] * sw_ref[0]

        def a_idx(j, d, h):
            return j * 4 + d * 2 + h

        def a_c0(j, d, h):
            return d * half + j * (2 * SUB) + h * SUB

        gf_seed = lax.rem(q + P - 1, P)
        gb_seed = lax.rem(q + 1, P)
        dests = (pnext, pprev)

        for m in range(N_MACRO + P - 1):
            if m < N_MACRO:
                j = m
                dfs = pgroup(gf_seed, a_c0(j, 0, 0), 2 * SUB)
                dbs = pgroup(gb_seed, a_c0(j, 1, 0), 2 * SUB)
                for d, dd in ((0, dfs), (1, dbs)):
                    for h in range(2):
                        comm_a[a_idx(j, d, h), 0, :, :] = (
                            dd[:, h * SUB:(h + 1) * SUB].astype(jnp.bfloat16)
                        )
                        mk_a(a_idx(j, d, h), 0, dests[d]).start()
                for s in range(P - 1):
                    gf = lax.rem(q + 2 * P - 2 - s, P)
                    gb = lax.rem(q + 2 + s, P)
                    df = pgroup(gf, a_c0(j, 0, 0), 2 * SUB)
                    db = pgroup(gb, a_c0(j, 1, 0), 2 * SUB)
                    for d, dd in ((0, df), (1, db)):
                        for h in range(2):
                            idx = a_idx(j, d, h)
                            part = dd[:, h * SUB:(h + 1) * SUB]
                            hdl = mk_a(idx, s, dests[d])
                            hdl.wait()
                            if s < P - 2:
                                comm_a[idx, s + 1, :, :] = (
                                    comm_a[idx, s + 1, :, :].astype(jnp.float32)
                                    + part
                                ).astype(jnp.bfloat16)
                                mk_a(idx, s + 1, dests[d]).start()
                            else:
                                stage[j * 2 + d, :, h * SUB:(h + 1) * SUB] = (
                                    comm_a[idx, P - 1, :, :].astype(jnp.float32)
                                    + part
                                )
                for t8 in (j * 2, j * 2 + 1):
                    @pl.when(r == 0)
                    def _(t8=t8):
                        for o in range(1, P):
                            in_up[t8, o, :, :] = (
                                stage[t8, o * BLK:(o + 1) * BLK, :]
                                .astype(jnp.bfloat16)
                            )
                            mk_up(t8, o).start()

                    @pl.when(r == P - 1)
                    def _(t8=t8):
                        for o in range(P - 1):
                            in_dn[t8, o, :, :] = (
                                stage[t8, o * BLK:(o + 1) * BLK, :]
                                .astype(jnp.bfloat16)
                            )
                            mk_dn(t8, o).start()

            for dlt in range(1, P):
                j2 = m - dlt
                if not 0 <= j2 < N_MACRO:
                    continue
                for t8 in (j2 * 2, j2 * 2 + 1):
                    @pl.when(r == dlt)
                    def _(t8=t8, dlt=dlt):
                        for o in range(dlt, P):
                            mk_up(t8, o).wait_recv()
                            blk = stage[t8, o * BLK:(o + 1) * BLK, :]
                            if o > dlt:
                                in_up[t8, o, :, :] = (
                                    in_up[t8, o, :, :].astype(jnp.float32)
                                    + blk
                                ).astype(jnp.bfloat16)
                                mk_up(t8, o).start()
                            else:
                                stage[t8, o * BLK:(o + 1) * BLK, :] = (
                                    blk + in_up[t8, o, :, :].astype(jnp.float32)
                                )

                    @pl.when(r == P - 1 - dlt)
                    def _(t8=t8, dlt=dlt):
                        for o in range(P - dlt):
                            mk_dn(t8, o).wait_recv()
                            blk = stage[t8, o * BLK:(o + 1) * BLK, :]
                            if o < P - 1 - dlt:
                                in_dn[t8, o, :, :] = (
                                    in_dn[t8, o, :, :].astype(jnp.float32)
                                    + blk
                                ).astype(jnp.bfloat16)
                                mk_dn(t8, o).start()
                            else:
                                stage[t8, o * BLK:(o + 1) * BLK, :] = (
                                    blk + in_dn[t8, o, :, :].astype(jnp.float32)
                                )

        for t8 in range(2 * N_MACRO):
            for o in range(1, P):
                @pl.when(r < o)
                def _(t8=t8, o=o):
                    mk_up(t8, o).wait_send()
            for o in range(P - 1):
                @pl.when(r > o)
                def _(t8=t8, o=o):
                    mk_dn(t8, o).wait_send()

        for j in range(N_MACRO):
            for d in range(2):
                t8 = j * 2 + d
                c0 = d * half + j * (2 * SUB)
                out_ref[:, c0:c0 + 2 * SUB] = (
                    stage[t8, pl.ds(r * BLK, BLK), :] * scale
                )

    return pl.pallas_call(
        body,
        out_shape=jax.ShapeDtypeStruct((m_per, n), jnp.float32),
        in_specs=[
            pl.BlockSpec(memory_space=pltpu.VMEM),
            pl.BlockSpec(memory_space=pltpu.VMEM),
            pl.BlockSpec(memory_space=pltpu.SMEM),
            pl.BlockSpec(memory_space=pltpu.SMEM),
        ],
        out_specs=pl.BlockSpec(memory_space=pltpu.VMEM),
        scratch_shapes=[
            pltpu.VMEM((m_total, k), jnp.float32),
            pltpu.VMEM((16, P, g_rows, SUB), jnp.bfloat16),
            pltpu.VMEM((8, g_rows, 2 * SUB), jnp.float32),
            pltpu.VMEM((8, P, BLK, 2 * SUB), jnp.bfloat16),
            pltpu.VMEM((8, P, BLK, 2 * SUB), jnp.bfloat16),
            pltpu.SemaphoreType.DMA((16, P - 1)),
            pltpu.SemaphoreType.DMA((16, P)),
            pltpu.SemaphoreType.DMA((8, P)),
            pltpu.SemaphoreType.DMA((8, P)),
            pltpu.SemaphoreType.DMA((8, P)),
            pltpu.SemaphoreType.DMA((8, P)),
        ],
        compiler_params=pltpu.CompilerParams(
            vmem_limit_bytes=100 * 1024 * 1024
        ),
    )(x, w_mat, scale_x, scale_w)


# device time: 113480 ns/iter; 1.0472x vs baseline; 1.0472x over previous
import jax
import jax.numpy as jnp
from jax import lax
from jax.experimental import pallas as pl
from jax.experimental.pallas import tpu as pltpu

N_DEV = 16
P = 4
N_MACRO = 4
SUB = 128
BLK = 256


def kernel(x, w_mat, scale_x, scale_w):
    m_total, k = x.shape
    _, n = w_mat.shape
    m_per = m_total // N_DEV
    half = n // 2
    g_rows = P * m_per

    def body(x_ref, w_ref, sx_ref, sw_ref, out_ref,
             xg_ref, comm_a, stage, in_up, in_dn,
             a_send, a_recv, up_send, up_recv, dn_send, dn_recv):
        my = lax.axis_index("i")
        q = lax.rem(my, P)
        r = my // P
        base = my - q
        pnext = base + lax.rem(q + 1, P)
        pprev = base + lax.rem(q + P - 1, P)
        zup = lax.rem(r + 1, P) * P + q
        zdn = lax.rem(r + P - 1, P) * P + q

        barrier_sem = pltpu.get_barrier_semaphore()
        for nbr in (pnext, pprev, zup, zdn):
            pl.semaphore_signal(
                barrier_sem, inc=1,
                device_id=(nbr,), device_id_type=pl.DeviceIdType.MESH,
            )
        pl.semaphore_wait(barrier_sem, 4)

        for g in range(P):
            for rb in range(P):
                c = P * rb + g
                xg_ref[pl.ds((g * P + rb) * m_per, m_per), :] = (
                    x_ref[pl.ds(c * m_per, m_per), :]
                )

        def pgroup(g, c0, width):
            return jnp.dot(
                xg_ref[pl.ds(g * g_rows, g_rows), :],
                w_ref[:, c0:c0 + width],
                preferred_element_type=jnp.float32,
            )

        def mk_a(idx, s, dest):
            return pltpu.make_async_remote_copy(
                src_ref=comm_a.at[idx, s],
                dst_ref=comm_a.at[idx, s + 1],
                send_sem=a_send.at[idx, s],
                recv_sem=a_recv.at[idx, s + 1],
                device_id=(dest,),
                device_id_type=pl.DeviceIdType.MESH,
            )

        def mk_up(t8, o):
            return pltpu.make_async_remote_copy(
                src_ref=in_up.at[t8, o], dst_ref=in_up.at[t8, o],
                send_sem=up_send.at[t8, o], recv_sem=up_recv.at[t8, o],
                device_id=(my + P,), device_id_type=pl.DeviceIdType.MESH,
            )

        def mk_dn(t8, o):
            return pltpu.make_async_remote_copy(
                src_ref=in_dn.at[t8, o], dst_ref=in_dn.at[t8, o],
                send_sem=dn_send.at[t8, o], recv_sem=dn_recv.at[t8, o],
                device_id=(my - P,), device_id_type=pl.DeviceIdType.MESH,
            )

        scale = sx_ref[0] * sw_ref[0]

        def a_idx(j, d, h):
            return j * 4 + d * 2 + h

        def a_c0(j, d, h):
            return d * half + j * (2 * SUB) + h * SUB

        gf_seed = lax.rem(q + P - 1, P)
        gb_seed = lax.rem(q + 1, P)
        dests = (pnext, pprev)

        def emit_seed(j):
            dfs = pgroup(gf_seed, a_c0(j, 0, 0), 2 * SUB)
            dbs = pgroup(gb_seed, a_c0(j, 1, 0), 2 * SUB)
            for d, dd in ((0, dfs), (1, dbs)):
                for h in range(2):
                    comm_a[a_idx(j, d, h), 0, :, :] = (
                        dd[:, h * SUB:(h + 1) * SUB].astype(jnp.bfloat16)
                    )
                    mk_a(a_idx(j, d, h), 0, dests[d]).start()

        emit_seed(0)

        for m in range(N_MACRO + P - 1):
            if m < N_MACRO:
                j = m
                for s in range(P - 1):
                    gf = lax.rem(q + 2 * P - 2 - s, P)
                    gb = lax.rem(q + 2 + s, P)
                    df = pgroup(gf, a_c0(j, 0, 0), 2 * SUB)
                    db = pgroup(gb, a_c0(j, 1, 0), 2 * SUB)
                    for d, dd in ((0, df), (1, db)):
                        for h in range(2):
                            idx = a_idx(j, d, h)
                            part = dd[:, h * SUB:(h + 1) * SUB]
                            hdl = mk_a(idx, s, dests[d])
                            hdl.wait()
                            if s < P - 2:
                                comm_a[idx, s + 1, :, :] = (
                                    comm_a[idx, s + 1, :, :].astype(jnp.float32)
                                    + part
                                ).astype(jnp.bfloat16)
                                mk_a(idx, s + 1, dests[d]).start()
                            else:
                                stage[j * 2 + d, :, h * SUB:(h + 1) * SUB] = (
                                    comm_a[idx, P - 1, :, :].astype(jnp.float32)
                                    + part
                                )
                    if s == 0 and j + 1 < N_MACRO:
                        emit_seed(j + 1)
                for t8 in (j * 2, j * 2 + 1):
                    @pl.when(r == 0)
                    def _(t8=t8):
                        for o in range(1, P):
                            in_up[t8, o, :, :] = (
                                stage[t8, o * BLK:(o + 1) * BLK, :]
                                .astype(jnp.bfloat16)
                            )
                            mk_up(t8, o).start()

                    @pl.when(r == P - 1)
                    def _(t8=t8):
                        for o in range(P - 1):
                            in_dn[t8, o, :, :] = (
                                stage[t8, o * BLK:(o + 1) * BLK, :]
                                .astype(jnp.bfloat16)
                            )
                            mk_dn(t8, o).start()

            for dlt in range(1, P):
                j2 = m - dlt
                if not 0 <= j2 < N_MACRO:
                    continue
                for t8 in (j2 * 2, j2 * 2 + 1):
                    @pl.when(r == dlt)
                    def _(t8=t8, dlt=dlt):
                        for o in range(dlt, P):
                            mk_up(t8, o).wait_recv()
                            blk = stage[t8, o * BLK:(o + 1) * BLK, :]
                            if o > dlt:
                                in_up[t8, o, :, :] = (
                                    in_up[t8, o, :, :].astype(jnp.float32)
                                    + blk
                                ).astype(jnp.bfloat16)
                                mk_up(t8, o).start()
                            else:
                                stage[t8, o * BLK:(o + 1) * BLK, :] = (
                                    blk + in_up[t8, o, :, :].astype(jnp.float32)
                                )

                    @pl.when(r == P - 1 - dlt)
                    def _(t8=t8, dlt=dlt):
                        for o in range(P - dlt):
                            mk_dn(t8, o).wait_recv()
                            blk = stage[t8, o * BLK:(o + 1) * BLK, :]
                            if o < P - 1 - dlt:
                                in_dn[t8, o, :, :] = (
                                    in_dn[t8, o, :, :].astype(jnp.float32)
                                    + blk
                                ).astype(jnp.bfloat16)
                                mk_dn(t8, o).start()
                            else:
                                stage[t8, o * BLK:(o + 1) * BLK, :] = (
                                    blk + in_dn[t8, o, :, :].astype(jnp.float32)
                                )

        for t8 in range(2 * N_MACRO):
            for o in range(1, P):
                @pl.when(r < o)
                def _(t8=t8, o=o):
                    mk_up(t8, o).wait_send()
            for o in range(P - 1):
                @pl.when(r > o)
                def _(t8=t8, o=o):
                    mk_dn(t8, o).wait_send()

        for j in range(N_MACRO):
            for d in range(2):
                t8 = j * 2 + d
                c0 = d * half + j * (2 * SUB)
                out_ref[:, c0:c0 + 2 * SUB] = (
                    stage[t8, pl.ds(r * BLK, BLK), :] * scale
                )

    return pl.pallas_call(
        body,
        out_shape=jax.ShapeDtypeStruct((m_per, n), jnp.float32),
        in_specs=[
            pl.BlockSpec(memory_space=pltpu.VMEM),
            pl.BlockSpec(memory_space=pltpu.VMEM),
            pl.BlockSpec(memory_space=pltpu.SMEM),
            pl.BlockSpec(memory_space=pltpu.SMEM),
        ],
        out_specs=pl.BlockSpec(memory_space=pltpu.VMEM),
        scratch_shapes=[
            pltpu.VMEM((m_total, k), jnp.float32),
            pltpu.VMEM((16, P, g_rows, SUB), jnp.bfloat16),
            pltpu.VMEM((8, g_rows, 2 * SUB), jnp.float32),
            pltpu.VMEM((8, P, BLK, 2 * SUB), jnp.bfloat16),
            pltpu.VMEM((8, P, BLK, 2 * SUB), jnp.bfloat16),
            pltpu.SemaphoreType.DMA((16, P - 1)),
            pltpu.SemaphoreType.DMA((16, P)),
            pltpu.SemaphoreType.DMA((8, P)),
            pltpu.SemaphoreType.DMA((8, P)),
            pltpu.SemaphoreType.DMA((8, P)),
            pltpu.SemaphoreType.DMA((8, P)),
        ],
        compiler_params=pltpu.CompilerParams(
            collective_id=0, vmem_limit_bytes=100 * 1024 * 1024
        ),
    )(x, w_mat, scale_x, scale_w)


# device time: 100508 ns/iter; 1.1824x vs baseline; 1.1291x over previous
import jax
import jax.numpy as jnp
from jax import lax
from jax.experimental import pallas as pl
from jax.experimental.pallas import tpu as pltpu

N_DEV = 16
N_STREAMS = 8
PER_DIR = N_STREAMS // 2


def kernel(x, w_mat, scale_x, scale_w):
    m_total, k = x.shape
    _, n = w_mat.shape
    m_per = m_total // N_DEV
    half = n // 2
    sub = n // N_STREAMS

    def body(x_ref, w_ref, sx_ref, sw_ref, out_ref, comm_ref, send_sems, recv_sems):
        my = lax.axis_index("i")
        left = lax.rem(my + N_DEV - 1, N_DEV)
        right = lax.rem(my + 1, N_DEV)

        barrier_sem = pltpu.get_barrier_semaphore()
        for nbr in (left, right):
            pl.semaphore_signal(
                barrier_sem, inc=1,
                device_id=(nbr,), device_id_type=pl.DeviceIdType.MESH,
            )
        pl.semaphore_wait(barrier_sem, 2)

        def xs(c):
            return x_ref[pl.ds(c * m_per, m_per), :]

        def pfwd(c):
            return jnp.dot(xs(c), w_ref[:, 0:half],
                           preferred_element_type=jnp.float32)

        def pbwd(c):
            return jnp.dot(xs(c), w_ref[:, half:n],
                           preferred_element_type=jnp.float32)

        def psub(c, k_):
            j0 = k_ * sub
            return jnp.dot(xs(c), w_ref[:, j0:j0 + sub],
                           preferred_element_type=jnp.float32)

        dests = tuple([right] * PER_DIR + [left] * PER_DIR)

        def mk(k_, s_):
            return pltpu.make_async_remote_copy(
                src_ref=comm_ref.at[k_, s_],
                dst_ref=comm_ref.at[k_, s_ + 1],
                send_sem=send_sems.at[k_, s_],
                recv_sem=recv_sems.at[k_, s_ + 1],
                device_id=(dests[k_],),
                device_id_type=pl.DeviceIdType.MESH,
            )

        for k_ in (0, 4, 1, 5, 2, 6, 3, 7):
            c_seed = left if k_ < PER_DIR else right
            comm_ref[k_, 0, :, :] = psub(c_seed, k_).astype(jnp.bfloat16)
            mk(k_, 0).start()

        scale = sx_ref[0] * sw_ref[0]

        for s in range(N_DEV - 1):
            cf = lax.rem(my + 2 * N_DEV - 2 - s, N_DEV)
            cb = lax.rem(my + 2 + s, N_DEV)
            pf = pfwd(cf)
            pb = pbwd(cb)
            parts = tuple(
                pf[:, k_ * sub:(k_ + 1) * sub] for k_ in range(PER_DIR)
            ) + tuple(
                pb[:, k_ * sub:(k_ + 1) * sub] for k_ in range(PER_DIR)
            )
            for k_ in (0, 4, 1, 5, 2, 6, 3, 7):
                h = mk(k_, s)
                h.wait()
                acc = comm_ref[k_, s + 1, :, :].astype(jnp.float32) + parts[k_]
                if s < N_DEV - 2:
                    comm_ref[k_, s + 1, :, :] = acc.astype(jnp.bfloat16)
                    mk(k_, s + 1).start()
                else:
                    col0 = k_ * sub
                    out_ref[:, col0:col0 + sub] = acc * scale

    return pl.pallas_call(
        body,
        out_shape=jax.ShapeDtypeStruct((m_per, n), jnp.float32),
        in_specs=[
            pl.BlockSpec(memory_space=pltpu.VMEM),
            pl.BlockSpec(memory_space=pltpu.VMEM),
            pl.BlockSpec(memory_space=pltpu.SMEM),
            pl.BlockSpec(memory_space=pltpu.SMEM),
        ],
        out_specs=pl.BlockSpec(memory_space=pltpu.VMEM),
        scratch_shapes=[
            pltpu.VMEM((N_STREAMS, N_DEV, m_per, sub), jnp.bfloat16),
            pltpu.SemaphoreType.DMA((N_STREAMS, N_DEV)),
            pltpu.SemaphoreType.DMA((N_STREAMS, N_DEV)),
        ],
        compiler_params=pltpu.CompilerParams(
            collective_id=0, vmem_limit_bytes=100 * 1024 * 1024
        ),
    )(x, w_mat, scale_x, scale_w)
